# baseline (speedup 1.0000x reference)
import functools

import jax
import jax.numpy as jnp
from jax import lax
from jax.experimental import pallas as pl
from jax.experimental.pallas import tpu as pltpu
from jax.experimental.pallas import tpu_sc as plsc

C_EMB = 64
NXY = 512 * 512
N_POINTS = 100000
D = 8

NC = 2
NS = 16
NW = NC * NS
CHUNK = 128
NCHUNK = 25
BPW = CHUNK * NCHUNK
NPAD = NW * BPW
L = 16

XB = 8
BN3 = 2048


def _proj_body(e_ref, a_ref, m_ref, o_ref):
    m = m_ref[...]
    for xx in range(XB):
        y_rep = lax.dot_general(
            e_ref[:, xx, :], a_ref[...],
            dimension_numbers=(((0,), (0,)), ((), ())),
            preferred_element_type=jnp.float32,
        )
        y3 = y_rep.reshape(32, 16, 128)
        o_ref[xx, :, :] = jnp.sum(y3 * m[None, :, :], axis=1)


def _project_table(emb3d, a_rep, mask):
    return pl.pallas_call(
        _proj_body,
        grid=(512 // XB,),
        in_specs=[
            pl.BlockSpec((C_EMB, XB, 512), lambda i: (0, i, 0)),
            pl.BlockSpec((C_EMB, 128), lambda i: (0, 0)),
            pl.BlockSpec((16, 128), lambda i: (0, 0)),
        ],
        out_specs=pl.BlockSpec((XB, 32, 128), lambda i: (i, 0, 0)),
        out_shape=jax.ShapeDtypeStruct((512, 32, 128), jnp.float32),
    )(emb3d, a_rep, mask)


def _sc_gather_body(p_hbm, idx_hbm, g_hbm, idx_v, rows_v, sem):
    wid = lax.axis_index("s") * NC + lax.axis_index("c")
    base = wid * BPW
    pltpu.sync_copy(idx_hbm.at[pl.ds(base, BPW)], idx_v)
    copies = []
    for j in range(NCHUNK):
        copies.append(pltpu.async_copy(
            p_hbm.at[idx_v.at[pl.ds(j * CHUNK, CHUNK)]],
            rows_v.at[pl.ds(j * CHUNK, CHUNK), :],
            sem,
        ))
    for cp in copies:
        cp.wait()
    pltpu.sync_copy(rows_v, g_hbm.at[pl.ds(base, BPW)])


def _sc_gather(p_flat, idx_pad):
    mesh = plsc.VectorSubcoreMesh(core_axis_name="c", subcore_axis_name="s")
    kern = functools.partial(
        pl.kernel,
        mesh=mesh,
        out_type=jax.ShapeDtypeStruct((NPAD, D), jnp.float32),
        scratch_types=[
            pltpu.VMEM((BPW,), jnp.int32),
            pltpu.VMEM((BPW, D), jnp.float32),
            pltpu.SemaphoreType.DMA,
        ],
        compiler_params=pltpu.CompilerParams(use_tc_tiling_on_sc=False),
    )(_sc_gather_body)
    return kern(p_flat.reshape(NXY, D), idx_pad)


def _head_body(pct_ref, g_ref, b_ref, m_ref, c_ref, o_ref):
    y_rep = lax.dot_general(
        pct_ref[...], b_ref[...],
        dimension_numbers=(((0,), (0,)), ((), ())),
        preferred_element_type=jnp.float32,
    )
    y3 = y_rep.reshape(BN3 // 16, 16, 128)
    y_packed = jnp.sum(y3 * m_ref[...][None, :, :], axis=1)
    o_ref[...] = y_packed + g_ref[...] + c_ref[...]


def _head(pct, g128, b_rep, mask, c_rep):
    grid = (pl.cdiv(N_POINTS, BN3),)
    return pl.pallas_call(
        _head_body,
        grid=grid,
        in_specs=[
            pl.BlockSpec((C_EMB, BN3), lambda i: (0, i)),
            pl.BlockSpec((BN3 // 16, 128), lambda i: (i, 0)),
            pl.BlockSpec((C_EMB, 128), lambda i: (0, 0)),
            pl.BlockSpec((16, 128), lambda i: (0, 0)),
            pl.BlockSpec((1, 128), lambda i: (0, 0)),
        ],
        out_specs=pl.BlockSpec((BN3 // 16, 128), lambda i: (i, 0)),
        out_shape=jax.ShapeDtypeStruct((NPAD // 16, 128), jnp.float32),
    )(pct, g128, b_rep, mask, c_rep)


def kernel(grid_flow_embeddings, point_cloud, grid_indices, WY, bY, WZ, bZ):
    a = (WZ @ WY[:, :C_EMB]).T
    b = (WZ @ WY[:, C_EMB:]).T
    c = WZ @ bY + bZ
    a_pad = jnp.pad(a, ((0, 0), (0, D - 3)))
    a_rep = jnp.tile(a_pad, (1, 16))
    sel = jnp.repeat(jnp.eye(16, dtype=jnp.float32), D, axis=1)
    b_pad = jnp.pad(b, ((0, 0), (0, D - 3)))
    b_rep = jnp.tile(b_pad, (1, 16))
    c_rep = jnp.tile(jnp.pad(c, (0, D - 3)), 16).reshape(1, 128)
    idx_pad = jnp.pad(grid_indices.astype(jnp.int32), (0, NPAD - N_POINTS))

    p = _project_table(grid_flow_embeddings, a_rep, sel)
    g = _sc_gather(p, idx_pad)
    out_packed = _head(point_cloud.T, g.reshape(NPAD // 16, 128), b_rep, sel,
                       c_rep)
    return out_packed.reshape(NPAD, D)[:N_POINTS, :3]


# speedup vs baseline: 1.3430x; 1.3430x over previous
import functools

import jax
import jax.numpy as jnp
from jax import lax
from jax.experimental import pallas as pl
from jax.experimental.pallas import tpu as pltpu
from jax.experimental.pallas import tpu_sc as plsc

C_EMB = 64
NXY = 512 * 512
N_POINTS = 100000
D = 8

NC = 2
NS = 16
NW = NC * NS
CHUNK = 128
NCHUNK = 25
BPW = CHUNK * NCHUNK
NPAD = NW * BPW
L = 16

XB = 8
BN3 = 2048


def _proj_body(e_ref, a_ref, m_ref, o_ref):
    m = m_ref[...]
    for xx in range(XB):
        y_rep = lax.dot_general(
            e_ref[:, xx, :], a_ref[...],
            dimension_numbers=(((0,), (0,)), ((), ())),
            preferred_element_type=jnp.float32,
        )
        y3 = y_rep.reshape(32, 16, 128)
        o_ref[xx, :, :] = jnp.sum(y3 * m[None, :, :], axis=1)


def _project_table(emb3d, a_rep, mask):
    return pl.pallas_call(
        _proj_body,
        grid=(512 // XB,),
        in_specs=[
            pl.BlockSpec((C_EMB, XB, 512), lambda i: (0, i, 0)),
            pl.BlockSpec((C_EMB, 128), lambda i: (0, 0)),
            pl.BlockSpec((16, 128), lambda i: (0, 0)),
        ],
        out_specs=pl.BlockSpec((XB, 32, 128), lambda i: (i, 0, 0)),
        out_shape=jax.ShapeDtypeStruct((512, 32, 128), jnp.float32),
    )(emb3d, a_rep, mask)


def _sc_gather_body(p_hbm, idx_hbm, g0_hbm, g1_hbm, g2_hbm,
                    idx_v, idxe_v, gt_v, sem):
    wid = lax.axis_index("s") * NC + lax.axis_index("c")
    base = wid * BPW
    pltpu.sync_copy(idx_hbm.at[pl.ds(base, BPW)], idx_v)

    for ch, g_hbm in ((0, g0_hbm), (1, g1_hbm), (2, g2_hbm)):
        def build(i, carry, ch=ch):
            v = idx_v[pl.ds(i * L, L)]
            idxe_v[pl.ds(i * L, L)] = v * D + ch
            return carry

        lax.fori_loop(0, BPW // L, build, 0)
        copies = []
        for j in range(NCHUNK):
            copies.append(pltpu.async_copy(
                p_hbm.at[idxe_v.at[pl.ds(j * CHUNK, CHUNK)]],
                gt_v.at[pl.ds(j * CHUNK, CHUNK)],
                sem,
            ))
        for cp in copies:
            cp.wait()
        pltpu.sync_copy(gt_v, g_hbm.at[pl.ds(base, BPW)])


def _sc_gather(p_flat1d, idx_pad):
    mesh = plsc.VectorSubcoreMesh(core_axis_name="c", subcore_axis_name="s")
    kern = functools.partial(
        pl.kernel,
        mesh=mesh,
        out_type=[jax.ShapeDtypeStruct((NPAD,), jnp.float32)] * 3,
        scratch_types=[
            pltpu.VMEM((BPW,), jnp.int32),
            pltpu.VMEM((BPW,), jnp.int32),
            pltpu.VMEM((BPW,), jnp.float32),
            pltpu.SemaphoreType.DMA,
        ],
        compiler_params=pltpu.CompilerParams(use_tc_tiling_on_sc=False),
    )(_sc_gather_body)
    return kern(p_flat1d, idx_pad)


def _head_body(pct_ref, g0_ref, g1_ref, g2_ref, b_ref, c_ref,
               o0_ref, o1_ref, o2_ref):
    y = lax.dot_general(
        b_ref[...], pct_ref[...],
        dimension_numbers=(((0,), (0,)), ((), ())),
        preferred_element_type=jnp.float32,
    )
    o0_ref[...] = y[0, :] + g0_ref[...] + c_ref[0, 0]
    o1_ref[...] = y[1, :] + g1_ref[...] + c_ref[0, 1]
    o2_ref[...] = y[2, :] + g2_ref[...] + c_ref[0, 2]


def _head(pct, g0, g1, g2, b_pad, c_pad):
    grid = (pl.cdiv(N_POINTS, BN3),)
    out1d = jax.ShapeDtypeStruct((N_POINTS,), jnp.float32)
    g_spec = pl.BlockSpec((BN3,), lambda i: (i,))
    return pl.pallas_call(
        _head_body,
        grid=grid,
        in_specs=[
            pl.BlockSpec((C_EMB, BN3), lambda i: (0, i)),
            g_spec, g_spec, g_spec,
            pl.BlockSpec((C_EMB, D), lambda i: (0, 0)),
            pl.BlockSpec((1, D), lambda i: (0, 0)),
        ],
        out_specs=[g_spec, g_spec, g_spec],
        out_shape=[out1d, out1d, out1d],
    )(pct, g0, g1, g2, b_pad, c_pad)


def kernel(grid_flow_embeddings, point_cloud, grid_indices, WY, bY, WZ, bZ):
    a = (WZ @ WY[:, :C_EMB]).T
    b = (WZ @ WY[:, C_EMB:]).T
    c = WZ @ bY + bZ
    a_pad = jnp.pad(a, ((0, 0), (0, D - 3)))
    a_rep = jnp.tile(a_pad, (1, 16))
    sel = jnp.repeat(jnp.eye(16, dtype=jnp.float32), D, axis=1)
    b_pad = jnp.pad(b, ((0, 0), (0, D - 3)))
    b_rep = jnp.tile(b_pad, (1, 16))
    c_pad2 = jnp.pad(c, (0, D - 3)).reshape(1, D)
    idx_pad = jnp.pad(grid_indices.astype(jnp.int32), (0, NPAD - N_POINTS))

    p = _project_table(grid_flow_embeddings, a_rep, sel)
    g0, g1, g2 = _sc_gather(p.reshape(NXY * D), idx_pad)
    o0, o1, o2 = _head(point_cloud.T, g0, g1, g2, b_pad, c_pad2)
    return jnp.stack([o0, o1, o2], axis=1)


# speedup vs baseline: 1.3755x; 1.0242x over previous
import functools

import jax
import jax.numpy as jnp
from jax import lax
from jax.experimental import pallas as pl
from jax.experimental.pallas import tpu as pltpu
from jax.experimental.pallas import tpu_sc as plsc

C_EMB = 64
NXY = 512 * 512
N_POINTS = 100000
D = 8

NC = 2
NS = 16
NW = NC * NS
CHUNK = 128
NCHUNK = 25
BPW = CHUNK * NCHUNK
NPAD = NW * BPW
L = 16

XB = 8
BN3 = 2048


def _proj_body(e_ref, a_ref, m_ref, o_ref):
    m = m_ref[...]
    for xx in range(XB):
        y_rep = lax.dot_general(
            e_ref[:, xx, :], a_ref[...],
            dimension_numbers=(((0,), (0,)), ((), ())),
            preferred_element_type=jnp.float32,
        )
        y3 = y_rep.reshape(32, 16, 128)
        o_ref[xx, :, :] = jnp.sum(y3 * m[None, :, :], axis=1)


def _project_table(emb3d, a_rep, mask):
    return pl.pallas_call(
        _proj_body,
        grid=(512 // XB,),
        in_specs=[
            pl.BlockSpec((C_EMB, XB, 512), lambda i: (0, i, 0)),
            pl.BlockSpec((C_EMB, 128), lambda i: (0, 0)),
            pl.BlockSpec((16, 128), lambda i: (0, 0)),
        ],
        out_specs=pl.BlockSpec((XB, 32, 128), lambda i: (i, 0, 0)),
        out_shape=jax.ShapeDtypeStruct((512, 32, 128), jnp.float32),
    )(emb3d, a_rep, mask)


def _sc_gather_body(p_hbm, idx_hbm, g0_hbm, g1_hbm, g2_hbm,
                    idx_v, idxe_v, gt_v, sem):
    wid = lax.axis_index("s") * NC + lax.axis_index("c")
    base = wid * BPW
    pltpu.sync_copy(idx_hbm.at[pl.ds(base, BPW)], idx_v)

    for ch, g_hbm in ((0, g0_hbm), (1, g1_hbm), (2, g2_hbm)):
        def build(i, carry, ch=ch):
            v = idx_v[pl.ds(i * L, L)]
            idxe_v[pl.ds(i * L, L)] = v * D + ch
            return carry

        lax.fori_loop(0, BPW // L, build, 0)
        copies = []
        for j in range(NCHUNK):
            copies.append(pltpu.async_copy(
                p_hbm.at[idxe_v.at[pl.ds(j * CHUNK, CHUNK)]],
                gt_v.at[pl.ds(j * CHUNK, CHUNK)],
                sem,
            ))
        for cp in copies:
            cp.wait()
        pltpu.sync_copy(gt_v, g_hbm.at[pl.ds(base, BPW)])


def _sc_gather(p_flat1d, idx_pad):
    mesh = plsc.VectorSubcoreMesh(core_axis_name="c", subcore_axis_name="s")
    kern = functools.partial(
        pl.kernel,
        mesh=mesh,
        out_type=[jax.ShapeDtypeStruct((NPAD,), jnp.float32)] * 3,
        scratch_types=[
            pltpu.VMEM((BPW,), jnp.int32),
            pltpu.VMEM((BPW,), jnp.int32),
            pltpu.VMEM((BPW,), jnp.float32),
            pltpu.SemaphoreType.DMA,
        ],
        compiler_params=pltpu.CompilerParams(use_tc_tiling_on_sc=False),
    )(_sc_gather_body)
    return kern(p_flat1d, idx_pad)


def _dense_body(pct_ref, b_ref, c_ref, q0_ref, q1_ref, q2_ref):
    y = lax.dot_general(
        b_ref[...], pct_ref[...],
        dimension_numbers=(((0,), (0,)), ((), ())),
        preferred_element_type=jnp.float32,
    )
    q0_ref[...] = y[0, :] + c_ref[0, 0]
    q1_ref[...] = y[1, :] + c_ref[0, 1]
    q2_ref[...] = y[2, :] + c_ref[0, 2]


def _dense(pct, b_pad, c_pad):
    grid = (pl.cdiv(N_POINTS, BN3),)
    out1d = jax.ShapeDtypeStruct((N_POINTS,), jnp.float32)
    g_spec = pl.BlockSpec((BN3,), lambda i: (i,))
    return pl.pallas_call(
        _dense_body,
        grid=grid,
        in_specs=[
            pl.BlockSpec((C_EMB, BN3), lambda i: (0, i)),
            pl.BlockSpec((C_EMB, D), lambda i: (0, 0)),
            pl.BlockSpec((1, D), lambda i: (0, 0)),
        ],
        out_specs=[g_spec, g_spec, g_spec],
        out_shape=[out1d, out1d, out1d],
    )(pct, b_pad, c_pad)


def _add_body(q0_ref, q1_ref, q2_ref, g0_ref, g1_ref, g2_ref,
              o0_ref, o1_ref, o2_ref):
    o0_ref[...] = q0_ref[...] + g0_ref[...]
    o1_ref[...] = q1_ref[...] + g1_ref[...]
    o2_ref[...] = q2_ref[...] + g2_ref[...]


def _add(q0, q1, q2, g0, g1, g2):
    grid = (pl.cdiv(N_POINTS, BN3),)
    out1d = jax.ShapeDtypeStruct((N_POINTS,), jnp.float32)
    g_spec = pl.BlockSpec((BN3,), lambda i: (i,))
    return pl.pallas_call(
        _add_body,
        grid=grid,
        in_specs=[g_spec] * 6,
        out_specs=[g_spec, g_spec, g_spec],
        out_shape=[out1d, out1d, out1d],
    )(q0, q1, q2, g0, g1, g2)


def kernel(grid_flow_embeddings, point_cloud, grid_indices, WY, bY, WZ, bZ):
    a = (WZ @ WY[:, :C_EMB]).T
    b = (WZ @ WY[:, C_EMB:]).T
    c = WZ @ bY + bZ
    a_pad = jnp.pad(a, ((0, 0), (0, D - 3)))
    a_rep = jnp.tile(a_pad, (1, 16))
    sel = jnp.repeat(jnp.eye(16, dtype=jnp.float32), D, axis=1)
    b_pad = jnp.pad(b, ((0, 0), (0, D - 3)))
    b_rep = jnp.tile(b_pad, (1, 16))
    c_pad2 = jnp.pad(c, (0, D - 3)).reshape(1, D)
    idx_pad = jnp.pad(grid_indices.astype(jnp.int32), (0, NPAD - N_POINTS))

    p = _project_table(grid_flow_embeddings, a_rep, sel)
    g0, g1, g2 = _sc_gather(p.reshape(NXY * D), idx_pad)
    q0, q1, q2 = _dense(point_cloud.T, b_pad, c_pad2)
    o0, o1, o2 = _add(q0, q1, q2, g0, g1, g2)
    return jnp.stack([o0, o1, o2], axis=1)


# speedup vs baseline: 1.5852x; 1.1525x over previous
import functools

import jax
import jax.numpy as jnp
from jax import lax
from jax.experimental import pallas as pl
from jax.experimental.pallas import tpu as pltpu
from jax.experimental.pallas import tpu_sc as plsc

C_EMB = 64
NXY = 512 * 512
N_POINTS = 100000
D = 8

NC = 2
NS = 16
NW = NC * NS
CHUNK = 128
NCHUNK = 25
BPW = CHUNK * NCHUNK
NPAD = NW * BPW
L = 16

XB = 8
BN3 = 2048


def _proj_body(e_ref, a_ref, m_ref, o_ref):
    m = m_ref[...]
    for xx in range(XB):
        y_rep = lax.dot_general(
            e_ref[:, xx, :], a_ref[...],
            dimension_numbers=(((0,), (0,)), ((), ())),
            preferred_element_type=jnp.float32,
        )
        y3 = y_rep.reshape(32, 16, 128)
        o_ref[xx, :, :] = jnp.sum(y3 * m[None, :, :], axis=1)


def _project_table(emb3d, a_rep, mask):
    return pl.pallas_call(
        _proj_body,
        grid=(512 // XB,),
        in_specs=[
            pl.BlockSpec((C_EMB, XB, 512), lambda i: (0, i, 0)),
            pl.BlockSpec((C_EMB, 128), lambda i: (0, 0)),
            pl.BlockSpec((16, 128), lambda i: (0, 0)),
        ],
        out_specs=pl.BlockSpec((XB, 32, 128), lambda i: (i, 0, 0)),
        out_shape=jax.ShapeDtypeStruct((512, 32, 128), jnp.float32),
    )(emb3d, a_rep, mask)


def _sc_gather_body(p_hbm, idx_hbm, g0_hbm, g1_hbm, g2_hbm,
                    idx_v, idxe0_v, idxe1_v, idxe2_v,
                    gt0_v, gt1_v, gt2_v, sem0, sem1, sem2):
    wid = lax.axis_index("s") * NC + lax.axis_index("c")
    base = wid * BPW
    pltpu.sync_copy(idx_hbm.at[pl.ds(base, BPW)], idx_v)

    def build(i, carry):
        v = idx_v[pl.ds(i * L, L)]
        v8 = v * D
        idxe0_v[pl.ds(i * L, L)] = v8
        idxe1_v[pl.ds(i * L, L)] = v8 + 1
        idxe2_v[pl.ds(i * L, L)] = v8 + 2
        return carry

    lax.fori_loop(0, BPW // L, build, 0)

    chans = ((idxe0_v, gt0_v, sem0, g0_hbm), (idxe1_v, gt1_v, sem1, g1_hbm),
             (idxe2_v, gt2_v, sem2, g2_hbm))

    def fire(idxe, gt, sem):
        return [pltpu.async_copy(
            p_hbm.at[idxe.at[pl.ds(j * CHUNK, CHUNK)]],
            gt.at[pl.ds(j * CHUNK, CHUNK)],
            sem,
        ) for j in range(NCHUNK)]

    pending = fire(*chans[0][:3])
    for k in range(3):
        if k + 1 < 3:
            nxt = fire(*chans[k + 1][:3])
        for cp in pending:
            cp.wait()
        pltpu.sync_copy(chans[k][1], chans[k][3].at[pl.ds(base, BPW)])
        if k + 1 < 3:
            pending = nxt


def _sc_gather(p_flat1d, idx_pad):
    mesh = plsc.VectorSubcoreMesh(core_axis_name="c", subcore_axis_name="s")
    kern = functools.partial(
        pl.kernel,
        mesh=mesh,
        out_type=[jax.ShapeDtypeStruct((NPAD,), jnp.float32)] * 3,
        scratch_types=[
            pltpu.VMEM((BPW,), jnp.int32),
            pltpu.VMEM((BPW,), jnp.int32),
            pltpu.VMEM((BPW,), jnp.int32),
            pltpu.VMEM((BPW,), jnp.int32),
            pltpu.VMEM((BPW,), jnp.float32),
            pltpu.VMEM((BPW,), jnp.float32),
            pltpu.VMEM((BPW,), jnp.float32),
            pltpu.SemaphoreType.DMA,
            pltpu.SemaphoreType.DMA,
            pltpu.SemaphoreType.DMA,
        ],
        compiler_params=pltpu.CompilerParams(use_tc_tiling_on_sc=False),
    )(_sc_gather_body)
    return kern(p_flat1d, idx_pad)


def _dense_body(pct_ref, b_ref, c_ref, q0_ref, q1_ref, q2_ref):
    y = lax.dot_general(
        b_ref[...], pct_ref[...],
        dimension_numbers=(((0,), (0,)), ((), ())),
        preferred_element_type=jnp.float32,
    )
    q0_ref[...] = y[0, :] + c_ref[0, 0]
    q1_ref[...] = y[1, :] + c_ref[0, 1]
    q2_ref[...] = y[2, :] + c_ref[0, 2]


def _dense(pct, b_pad, c_pad):
    grid = (pl.cdiv(N_POINTS, BN3),)
    out1d = jax.ShapeDtypeStruct((N_POINTS,), jnp.float32)
    g_spec = pl.BlockSpec((BN3,), lambda i: (i,))
    return pl.pallas_call(
        _dense_body,
        grid=grid,
        in_specs=[
            pl.BlockSpec((C_EMB, BN3), lambda i: (0, i)),
            pl.BlockSpec((C_EMB, D), lambda i: (0, 0)),
            pl.BlockSpec((1, D), lambda i: (0, 0)),
        ],
        out_specs=[g_spec, g_spec, g_spec],
        out_shape=[out1d, out1d, out1d],
    )(pct, b_pad, c_pad)


def _add_body(q0_ref, q1_ref, q2_ref, g0_ref, g1_ref, g2_ref,
              o0_ref, o1_ref, o2_ref):
    o0_ref[...] = q0_ref[...] + g0_ref[...]
    o1_ref[...] = q1_ref[...] + g1_ref[...]
    o2_ref[...] = q2_ref[...] + g2_ref[...]


BN_ADD = 16384


def _add(q0, q1, q2, g0, g1, g2):
    grid = (pl.cdiv(N_POINTS, BN_ADD),)
    out1d = jax.ShapeDtypeStruct((N_POINTS,), jnp.float32)
    g_spec = pl.BlockSpec((BN_ADD,), lambda i: (i,))
    return pl.pallas_call(
        _add_body,
        grid=grid,
        in_specs=[g_spec] * 6,
        out_specs=[g_spec, g_spec, g_spec],
        out_shape=[out1d, out1d, out1d],
    )(q0, q1, q2, g0, g1, g2)


def kernel(grid_flow_embeddings, point_cloud, grid_indices, WY, bY, WZ, bZ):
    a = (WZ @ WY[:, :C_EMB]).T
    b = (WZ @ WY[:, C_EMB:]).T
    c = WZ @ bY + bZ
    a_pad = jnp.pad(a, ((0, 0), (0, D - 3)))
    a_rep = jnp.tile(a_pad, (1, 16))
    sel = jnp.repeat(jnp.eye(16, dtype=jnp.float32), D, axis=1)
    b_pad = jnp.pad(b, ((0, 0), (0, D - 3)))
    b_rep = jnp.tile(b_pad, (1, 16))
    c_pad2 = jnp.pad(c, (0, D - 3)).reshape(1, D)
    idx_pad = jnp.pad(grid_indices.astype(jnp.int32), (0, NPAD - N_POINTS))

    p = _project_table(grid_flow_embeddings, a_rep, sel)
    g0, g1, g2 = _sc_gather(p.reshape(NXY * D), idx_pad)
    q0, q1, q2 = _dense(point_cloud.T, b_pad, c_pad2)
    o0, o1, o2 = _add(q0, q1, q2, g0, g1, g2)
    return jnp.stack([o0, o1, o2], axis=1)
